# baseline (speedup 1.0000x reference)
import functools

import jax
import jax.numpy as jnp
from jax import lax
from jax.experimental import pallas as pl

_K = 8


def _routing_body(x_ref, scale_ref, wq_ref, bq_ref, wk_ref, bk_ref,
                  wv_ref, bv_ref, o_ref, *, n, d, k):
    xb = x_ref[0]
    inv_s = 1.0 / (d ** 0.5)
    qT = (jnp.dot(wq_ref[...], xb, preferred_element_type=jnp.float32)
          + bq_ref[...]) * inv_s
    kT = jnp.dot(wk_ref[...], xb, preferred_element_type=jnp.float32) + bk_ref[...]
    vT = jnp.dot(wv_ref[...], xb, preferred_element_type=jnp.float32) + bv_ref[...]

    sim = lax.dot_general(qT, kT, (((0,), (0,)), ((), ())),
                          preferred_element_type=jnp.float32)
    row = lax.broadcasted_iota(jnp.int32, (n, n), 0)
    col = lax.broadcasted_iota(jnp.int32, (n, n), 1)
    sim = jnp.where(row == col, sim - 1e9, sim)

    work = sim
    m0 = None
    for it in range(k - 1):
        m = jnp.max(work, axis=1, keepdims=True)
        if it == 0:
            m0 = m
        work = jnp.where(work == m, -jnp.inf, work)
    m_last = jnp.max(work, axis=1, keepdims=True)

    e = jnp.where((work == -jnp.inf) | (work == m_last),
                  jnp.exp(sim - m0), 0.0)
    ones_row = jnp.ones((1, n), jnp.float32)
    denom = lax.dot_general(ones_row, e, (((1,), (1,)), ((), ())),
                            preferred_element_type=jnp.float32)
    comb = lax.dot_general(vT, e, (((1,), (1,)), ((), ())),
                           preferred_element_type=jnp.float32)
    o_ref[0] = xb + (scale_ref[0, 0] / denom) * comb


def kernel(x, scale, Wq, bq, Wk, bk, Wv, bv):
    B_, C_, H_, W_ = x.shape
    N = H_ * W_
    D_ = Wq.shape[0]
    xr = x.reshape(B_, C_, N)
    body = functools.partial(_routing_body, n=N, d=D_, k=_K)
    out = pl.pallas_call(
        body,
        grid=(B_,),
        in_specs=[
            pl.BlockSpec((1, C_, N), lambda b: (b, 0, 0)),
            pl.BlockSpec((1, 1), lambda b: (0, 0)),
            pl.BlockSpec((D_, C_), lambda b: (0, 0)),
            pl.BlockSpec((D_, 1), lambda b: (0, 0)),
            pl.BlockSpec((D_, C_), lambda b: (0, 0)),
            pl.BlockSpec((D_, 1), lambda b: (0, 0)),
            pl.BlockSpec((C_, C_), lambda b: (0, 0)),
            pl.BlockSpec((C_, 1), lambda b: (0, 0)),
        ],
        out_specs=pl.BlockSpec((1, C_, N), lambda b: (b, 0, 0)),
        out_shape=jax.ShapeDtypeStruct((B_, C_, N), jnp.float32),
    )(xr, scale.reshape(1, 1), Wq, bq.reshape(D_, 1), Wk, bk.reshape(D_, 1),
      Wv, bv.reshape(C_, 1))
    return out.reshape(B_, C_, H_, W_)


# speedup vs baseline: 1.0125x; 1.0125x over previous
import functools

import jax
import jax.numpy as jnp
from jax import lax
from jax.experimental import pallas as pl

_K = 8


def _routing_body(x_ref, scale_ref, wq_ref, bq_ref, wk_ref, bk_ref,
                  wv_ref, bv_ref, o_ref, *, n, d, k, bps):
    for sb in range(bps):
        _routing_one(x_ref, scale_ref, wq_ref, bq_ref, wk_ref, bk_ref,
                     wv_ref, bv_ref, o_ref, sb, n=n, d=d, k=k)


def _routing_one(x_ref, scale_ref, wq_ref, bq_ref, wk_ref, bk_ref,
                 wv_ref, bv_ref, o_ref, sb, *, n, d, k):
    xb = x_ref[sb]
    inv_s = 1.0 / (d ** 0.5)
    qT = (jnp.dot(wq_ref[...], xb, preferred_element_type=jnp.float32)
          + bq_ref[...]) * inv_s
    kT = jnp.dot(wk_ref[...], xb, preferred_element_type=jnp.float32) + bk_ref[...]
    vT = jnp.dot(wv_ref[...], xb, preferred_element_type=jnp.float32) + bv_ref[...]

    sim = lax.dot_general(qT, kT, (((0,), (0,)), ((), ())),
                          preferred_element_type=jnp.float32)
    row = lax.broadcasted_iota(jnp.int32, (n, n), 0)
    col = lax.broadcasted_iota(jnp.int32, (n, n), 1)
    sim = jnp.where(row == col, sim - 1e9, sim)

    work = sim
    m0 = None
    for it in range(k - 1):
        m = jnp.max(work, axis=1, keepdims=True)
        if it == 0:
            m0 = m
        work = jnp.where(work == m, -jnp.inf, work)
    m_last = jnp.max(work, axis=1, keepdims=True)

    e = jnp.where((work == -jnp.inf) | (work == m_last),
                  jnp.exp(sim - m0), 0.0)
    ones_row = jnp.ones((1, n), jnp.float32)
    denom = lax.dot_general(ones_row, e, (((1,), (1,)), ((), ())),
                            preferred_element_type=jnp.float32)
    comb = lax.dot_general(vT, e, (((1,), (1,)), ((), ())),
                           preferred_element_type=jnp.float32)
    o_ref[sb] = xb + (scale_ref[0, 0] / denom) * comb


def kernel(x, scale, Wq, bq, Wk, bk, Wv, bv):
    B_, C_, H_, W_ = x.shape
    N = H_ * W_
    D_ = Wq.shape[0]
    BPS = 2
    xr = x.reshape(B_, C_, N)
    body = functools.partial(_routing_body, n=N, d=D_, k=_K, bps=BPS)
    out = pl.pallas_call(
        body,
        grid=(B_ // BPS,),
        in_specs=[
            pl.BlockSpec((BPS, C_, N), lambda b: (b, 0, 0)),
            pl.BlockSpec((1, 1), lambda b: (0, 0)),
            pl.BlockSpec((D_, C_), lambda b: (0, 0)),
            pl.BlockSpec((D_, 1), lambda b: (0, 0)),
            pl.BlockSpec((D_, C_), lambda b: (0, 0)),
            pl.BlockSpec((D_, 1), lambda b: (0, 0)),
            pl.BlockSpec((C_, C_), lambda b: (0, 0)),
            pl.BlockSpec((C_, 1), lambda b: (0, 0)),
        ],
        out_specs=pl.BlockSpec((BPS, C_, N), lambda b: (b, 0, 0)),
        out_shape=jax.ShapeDtypeStruct((B_, C_, N), jnp.float32),
    )(xr, scale.reshape(1, 1), Wq, bq.reshape(D_, 1), Wk, bk.reshape(D_, 1),
      Wv, bv.reshape(C_, 1))
    return out.reshape(B_, C_, H_, W_)
